# Initial kernel scaffold; baseline (speedup 1.0000x reference)
import jax
import jax.numpy as jnp
from jax.experimental import pallas as pl


def kernel(embeds, edge_index, edge_vals):
    raise NotImplementedError("write your pallas kernel here")


# speedup vs baseline: 4.3995x; 4.3995x over previous
import functools

import jax
import jax.numpy as jnp
from jax import lax
from jax.experimental import pallas as pl
from jax.experimental.pallas import tpu as pltpu
from jax.experimental.pallas import tpu_sc as plsc

N = 10000
E = 320000
D = 128
NPAD = 10240
NC = 2
NS = 16
NW = NC * NS
EPT = E // NW
C = 80
NCHUNK = EPT // C
RPT = NPAD // NS
NUM_CAND = 500
BLK = 1024


def _make_spmm_kernel(p1: float, p2: float, use_num: bool):
  mesh = plsc.VectorSubcoreMesh(core_axis_name="c", subcore_axis_name="s")
  out_type = (
      jax.ShapeDtypeStruct((NC, NPAD, D), jnp.float32),
      jax.ShapeDtypeStruct((NW, NPAD), jnp.float32),
      jax.ShapeDtypeStruct((NW, NPAD), jnp.float32),
  )
  scratch = [
      pltpu.VMEM_SHARED((NPAD, D), jnp.float32),
      pltpu.VMEM((C,), jnp.int32),
      pltpu.VMEM((C,), jnp.int32),
      pltpu.VMEM((C,), jnp.float32),
      pltpu.VMEM((C,), jnp.float32),
      pltpu.VMEM((C,), jnp.float32),
      pltpu.VMEM((C, D), jnp.float32),
      pltpu.VMEM((NPAD,), jnp.float32),
      pltpu.VMEM((NPAD,), jnp.float32),
      pltpu.VMEM((NPAD,), jnp.float32),
      pltpu.SemaphoreType.DMA,
  ]

  @functools.partial(
      pl.kernel, mesh=mesh, out_type=out_type, scratch_types=scratch,
      compiler_params=pltpu.CompilerParams(needs_layout_passes=False))
  def spmm(x_hbm, numvec_hbm, src_hbm, dst_hbm, vals_hbm, u1_hbm, u2_hbm,
           z2_hbm, z1_hbm, emb_out, num_out, ord_out,
           acc_sh, src_v, dst_v, val_v, u1_v, u2_v, rows_v,
           numvec_v, accn_v, acco_v, sem):
    cid = lax.axis_index("c")
    sid = lax.axis_index("s")
    wid = sid * NC + cid

    rsl = pl.ds(sid * RPT, RPT)
    pltpu.sync_copy(z2_hbm.at[rsl], acc_sh.at[rsl])
    pltpu.sync_copy(z1_hbm, accn_v)
    pltpu.sync_copy(z1_hbm, acco_v)
    if use_num:
      pltpu.sync_copy(numvec_hbm, numvec_v)
    plsc.subcore_barrier()

    def chunk_body(t, carry):
      base = wid * EPT + t * C
      esl = pl.ds(base, C)
      pltpu.sync_copy(src_hbm.at[esl], src_v)
      pltpu.sync_copy(dst_hbm.at[esl], dst_v)
      pltpu.sync_copy(vals_hbm.at[esl], val_v)
      if p1 < 1.0:
        pltpu.sync_copy(u1_hbm.at[esl], u1_v)
      if p2 < 1.0:
        pltpu.sync_copy(u2_hbm.at[esl], u2_v)
      gather = pltpu.async_copy(x_hbm.at[src_v], rows_v, sem)

      def grp_body(g, carry2):
        gsl = pl.ds(g * 16, 16)
        v = val_v[gsl]
        one = jnp.ones((16,), jnp.float32)
        zero = jnp.zeros((16,), jnp.float32)
        if p1 < 1.0:
          v = v * jnp.where(u1_v[gsl] >= (1.0 - p1), one, zero)
        if p2 < 1.0:
          v = v * jnp.where(u2_v[gsl] >= (1.0 - p2), one, zero)
        val_v[gsl] = v
        d16 = dst_v[gsl]
        if use_num:
          s16 = src_v[gsl]
          gat = plsc.load_gather(numvec_v, [s16])
          plsc.addupdate_scatter(accn_v, [d16], gat * v)
        plsc.addupdate_scatter(acco_v, [d16], v)
        return carry2

      lax.fori_loop(0, C // 16, grp_body, 0)
      gather.wait()

      def scale_body(g, carry2):
        v16 = val_v[pl.ds(g * 16, 16)]
        for e in range(16):
          row = g * 16 + e
          vs = jnp.full((16,), v16[e], jnp.float32)
          for j in range(D // 16):
            fsl = pl.ds(j * 16, 16)
            rows_v[row, fsl] = rows_v[row, fsl] * vs
        return carry2

      lax.fori_loop(0, C // 16, scale_body, 0)

      pltpu.sync_copy(rows_v, acc_sh.at[dst_v], add=True)
      return carry

    lax.fori_loop(0, NCHUNK, chunk_body, 0)
    plsc.subcore_barrier()

    pltpu.sync_copy(acc_sh.at[rsl], emb_out.at[cid, rsl])
    pltpu.sync_copy(accn_v, num_out.at[wid])
    pltpu.sync_copy(acco_v, ord_out.at[wid])

  return spmm


_spmm_l0 = _make_spmm_kernel(1.0, 1.0, False)
_spmm_l1 = _make_spmm_kernel(0.5, 1.0, True)
_spmm_l2 = _make_spmm_kernel(0.5, 0.25, True)


def _combine_body(level, ep_ref, ns_ref, os_ref, pe_ref, pn_ref, po_ref,
                  emb_o, num_o, ord_o):
  sp_e = ep_ref[0] + ep_ref[1]
  sp_o = jnp.sum(os_ref[...], axis=0).reshape(BLK, 1)
  if level == 0:
    emb_o[...] = sp_e - pe_ref[...]
    num_o[...] = sp_o
  else:
    sp_n = jnp.sum(ns_ref[...], axis=0).reshape(BLK, 1)
    po = po_ref[...]
    emb_o[...] = sp_e - (1.0 + po) * pe_ref[...]
    num_o[...] = sp_n - pn_ref[...] - po
  ord_o[...] = sp_o


def _combine(level, emb_part, num_st, ord_st, prev_emb, prev_num, prev_ord):
  grid = (NPAD // BLK,)
  return pl.pallas_call(
      functools.partial(_combine_body, level),
      grid=grid,
      in_specs=[
          pl.BlockSpec((NC, BLK, D), lambda i: (0, i, 0)),
          pl.BlockSpec((NW, BLK), lambda i: (0, i)),
          pl.BlockSpec((NW, BLK), lambda i: (0, i)),
          pl.BlockSpec((BLK, D), lambda i: (i, 0)),
          pl.BlockSpec((BLK, 1), lambda i: (i, 0)),
          pl.BlockSpec((BLK, 1), lambda i: (i, 0)),
      ],
      out_specs=[
          pl.BlockSpec((BLK, D), lambda i: (i, 0)),
          pl.BlockSpec((BLK, 1), lambda i: (i, 0)),
          pl.BlockSpec((BLK, 1), lambda i: (i, 0)),
      ],
      out_shape=[
          jax.ShapeDtypeStruct((NPAD, D), jnp.float32),
          jax.ShapeDtypeStruct((NPAD, 1), jnp.float32),
          jax.ShapeDtypeStruct((NPAD, 1), jnp.float32),
      ],
  )(emb_part, num_st, ord_st, prev_emb, prev_num, prev_ord)


def _scores_body(e1, e2, e3, n1, n2, n3, emb, un, out):
  i = pl.program_id(0)
  sum_e = e1[...] + e2[...] + e3[...]
  sum_n = n1[...] + n2[...] + n3[...]
  sub = sum_e / (sum_n + 1e-08)
  nrm = jnp.sqrt(jnp.sum(sub * sub, axis=1, keepdims=True))
  sub = sub / jnp.maximum(nrm, 1e-12)
  eb = emb[...]
  enrm = jnp.sqrt(jnp.sum(eb * eb, axis=1, keepdims=True))
  eb = eb / jnp.maximum(enrm, 1e-12)
  dot = jnp.sum(sub * eb, axis=1, keepdims=True)
  noise = -jnp.log(-jnp.log(un[...]))
  rid = i * BLK + lax.broadcasted_iota(jnp.int32, (BLK, 1), 0)
  out[...] = jnp.where(rid < N, dot + noise, jnp.float32(-1e30))


def _scores(e1, e2, e3, n1, n2, n3, emb, un):
  grid = (NPAD // BLK,)
  bs_e = pl.BlockSpec((BLK, D), lambda i: (i, 0))
  bs_n = pl.BlockSpec((BLK, 1), lambda i: (i, 0))
  return pl.pallas_call(
      _scores_body,
      grid=grid,
      in_specs=[bs_e, bs_e, bs_e, bs_n, bs_n, bs_n, bs_e, bs_n],
      out_specs=bs_n,
      out_shape=jax.ShapeDtypeStruct((NPAD, 1), jnp.float32),
  )(e1, e2, e3, n1, n2, n3, emb, un)


def _topk_body(x_ref, out_ref):
  rows, cols = x_ref.shape
  ri = lax.broadcasted_iota(jnp.int32, (rows, cols), 0)
  ci = lax.broadcasted_iota(jnp.int32, (rows, cols), 1)
  gidx = ri * cols + ci

  def it(i, x):
    m = jnp.max(x)
    cand = jnp.where(x == m, gidx, jnp.int32(2**30))
    j = jnp.min(cand)
    out_ref[pl.ds(i, 1), :] = jnp.full((1, 1), j, jnp.int32)
    return jnp.where(gidx == j, jnp.float32(-3e38), x)

  lax.fori_loop(0, NUM_CAND, it, x_ref[...])


def _topk(scores_2d):
  return pl.pallas_call(
      _topk_body,
      out_shape=jax.ShapeDtypeStruct((NUM_CAND, 1), jnp.int32),
  )(scores_2d)


def kernel(embeds, edge_index, edge_vals):
  dst = edge_index[0]
  src = edge_index[1]

  key = jax.random.key(42)
  key, sub = jax.random.split(key)
  u1 = jax.random.uniform(sub, (E,), dtype=jnp.float32)
  key, sub = jax.random.split(key)
  u2 = jax.random.uniform(sub, (E,), dtype=jnp.float32)
  nkey = jax.random.key(7)
  un = jax.random.uniform(nkey, (N,), dtype=jnp.float32,
                          minval=1e-20, maxval=1.0)
  un_pad = jnp.pad(un, (0, NPAD - N), constant_values=0.5).reshape(NPAD, 1)

  embeds_pad = jnp.pad(embeds, ((0, NPAD - N), (0, 0)))
  z2 = jnp.zeros((NPAD, D), jnp.float32)
  z1 = jnp.zeros((NPAD,), jnp.float32)
  zcol = jnp.zeros((NPAD, 1), jnp.float32)

  ep0, _, os0 = _spmm_l0(embeds_pad, z1, src, dst, edge_vals, u1, u2, z2, z1)
  emb1, num1, ord0 = _combine(0, ep0, os0, os0, embeds_pad, zcol, zcol)

  ep1, ns1, os1 = _spmm_l1(emb1, num1[:, 0], src, dst, edge_vals, u1, u2,
                           z2, z1)
  emb2, num2, ord1 = _combine(1, ep1, ns1, os1, emb1, num1, ord0)

  ep2, ns2, os2 = _spmm_l2(emb2, num2[:, 0], src, dst, edge_vals, u1, u2,
                           z2, z1)
  emb3, num3, _ = _combine(1, ep2, ns2, os2, emb2, num2, ord1)

  scores_pad = _scores(emb1, emb2, emb3, num1, num2, num3, embeds_pad,
                       un_pad)
  cand = _topk(scores_pad.reshape(8, NPAD // 8))

  return scores_pad[:N, 0], cand[:, 0]


# speedup vs baseline: 9.5418x; 2.1688x over previous
import functools

import jax
import jax.numpy as jnp
from jax import lax
from jax.experimental import pallas as pl
from jax.experimental.pallas import tpu as pltpu
from jax.experimental.pallas import tpu_sc as plsc

N = 10000
E = 320000
D = 128
NPAD = 10240
NC = 2
NS = 16
NW = NC * NS
EPT = E // NW
C = 80
NCHUNK = EPT // C
RPT = NPAD // NS
NUM_CAND = 500
BLK = 1024

NROWBUF = 3
NEBUF = 6
G16 = C // 16


def _make_spmm_kernel(p1: float, p2: float, use_num: bool):
  mesh = plsc.VectorSubcoreMesh(core_axis_name="c", subcore_axis_name="s")
  out_type = (
      jax.ShapeDtypeStruct((NC, NPAD, D), jnp.float32),
      jax.ShapeDtypeStruct((NC, NPAD), jnp.float32),
      jax.ShapeDtypeStruct((NC, NPAD), jnp.float32),
  )
  scratch = (
      [pltpu.VMEM_SHARED((N, D), jnp.float32)]
      + [pltpu.VMEM_SHARED((NPAD,), jnp.float32)] * 2
      + [pltpu.VMEM((NPAD,), jnp.float32)]
      + [pltpu.VMEM((5, C), jnp.int32)] * NEBUF
      + [pltpu.VMEM((C,), jnp.float32)] * NEBUF
      + [pltpu.VMEM((C,), jnp.float32)] * NEBUF
      + [pltpu.VMEM((C, D), jnp.float32)] * NROWBUF
      + [pltpu.SemaphoreType.DMA] * (3 * NEBUF + 2 * NROWBUF)
  )

  @functools.partial(
      pl.kernel, mesh=mesh, out_type=out_type, scratch_types=scratch,
      compiler_params=pltpu.CompilerParams(needs_layout_passes=False))
  def spmm(x_hbm, numvec_hbm, edata_hbm, z2_hbm, z1_hbm,
           emb_out, num_out, ord_out,
           acc_sh, accn_sh, acco_sh, numvec_v, *rest):
    ebuf = rest[:NEBUF]
    vbuf = rest[NEBUF:2 * NEBUF]
    pbuf = rest[2 * NEBUF:3 * NEBUF]
    rows = rest[3 * NEBUF:3 * NEBUF + NROWBUF]
    sems = rest[3 * NEBUF + NROWBUF:]
    esem = sems[:NEBUF]
    vsem = sems[NEBUF:2 * NEBUF]
    psem = sems[2 * NEBUF:3 * NEBUF]
    gsem = sems[3 * NEBUF:3 * NEBUF + NROWBUF]
    ssem = sems[3 * NEBUF + NROWBUF:]
    cid = lax.axis_index("c")
    sid = lax.axis_index("s")
    wid = sid * NC + cid

    rsl = pl.ds(sid * RPT, RPT)
    lsl = pl.ds((NS - 1) * RPT, N - (NS - 1) * RPT)

    @pl.when(sid < NS - 1)
    def _():
      pltpu.sync_copy(z2_hbm.at[rsl], acc_sh.at[rsl])

    @pl.when(sid == NS - 1)
    def _():
      pltpu.sync_copy(z2_hbm.at[lsl], acc_sh.at[lsl])

    pltpu.sync_copy(z1_hbm.at[rsl], accn_sh.at[rsl])
    pltpu.sync_copy(z1_hbm.at[rsl], acco_sh.at[rsl])
    if use_num:
      pltpu.sync_copy(numvec_hbm, numvec_v)
    plsc.subcore_barrier()

    def scaled_process(t, k3, k6):
      def gbody(g, carry):
        gsl = pl.ds(g * 16, 16)
        v16 = plsc.bitcast(ebuf[k6][2, gsl], jnp.float32)
        one = jnp.ones((16,), jnp.float32)
        zero = jnp.zeros((16,), jnp.float32)
        if p1 < 1.0:
          u1v = plsc.bitcast(ebuf[k6][3, gsl], jnp.float32)
          v16 = v16 * jnp.where(u1v >= (1.0 - p1), one, zero)
        if p2 < 1.0:
          u2v = plsc.bitcast(ebuf[k6][4, gsl], jnp.float32)
          v16 = v16 * jnp.where(u2v >= (1.0 - p2), one, zero)
        vbuf[k6][gsl] = v16
        if use_num:
          s16 = ebuf[k6][0, gsl]
          g16 = plsc.load_gather(numvec_v, [s16])
          pbuf[k6][gsl] = g16 * v16
        for e in range(16):
          row = g * 16 + e
          vs = jnp.full((16,), v16[e], jnp.float32)
          for j in range(D // 16):
            fsl = pl.ds(j * 16, 16)
            rows[k3][row, fsl] = rows[k3][row, fsl] * vs
        return carry

      lax.fori_loop(0, G16, gbody, 0)

    def issue_scatters(t, k3, k6):
      idx = ebuf[k6].at[1]
      pltpu.async_copy(rows[k3], acc_sh.at[idx], ssem[k3], add=True)
      pltpu.async_copy(vbuf[k6], acco_sh.at[idx], vsem[k6], add=True)
      if use_num:
        pltpu.async_copy(pbuf[k6], accn_sh.at[idx], psem[k6], add=True)

    def step(u, k3, k6, *, e_wait_vp, e_issue, g_swait, g_issue):
      if e_issue:
        ke = (k6 + 4) % NEBUF
        if e_wait_vp:
          pltpu.make_async_copy(z1_hbm.at[pl.ds(0, C)], vbuf[ke],
                                vsem[ke]).wait()
          if use_num:
            pltpu.make_async_copy(z1_hbm.at[pl.ds(0, C)], pbuf[ke],
                                  psem[ke]).wait()
        pltpu.async_copy(edata_hbm.at[wid, u + 4], ebuf[ke], esem[ke])
      if g_issue:
        kg = (k3 + 2) % NROWBUF
        keg = (k6 + 2) % NEBUF
        pltpu.make_async_copy(edata_hbm.at[0, 0], ebuf[keg],
                              esem[keg]).wait()
        if g_swait:
          pltpu.make_async_copy(z2_hbm.at[pl.ds(0, C)], rows[kg],
                                ssem[kg]).wait()
        pltpu.async_copy(x_hbm.at[ebuf[keg].at[0]], rows[kg], gsem[kg])
      pltpu.make_async_copy(z2_hbm.at[pl.ds(0, C)], rows[k3],
                            gsem[k3]).wait()
      scaled_process(u, k3, k6)
      issue_scatters(u, k3, k6)

    for u0 in range(4):
      pltpu.async_copy(edata_hbm.at[wid, u0], ebuf[u0], esem[u0])
    for u0 in range(2):
      pltpu.make_async_copy(edata_hbm.at[0, 0], ebuf[u0], esem[u0]).wait()
      pltpu.async_copy(x_hbm.at[ebuf[u0].at[0]], rows[u0], gsem[u0])

    step(0, 0, 0, e_wait_vp=False, e_issue=True, g_swait=False, g_issue=True)
    step(1, 1, 1, e_wait_vp=False, e_issue=True, g_swait=True, g_issue=True)

    def main_body(i, carry):
      for k in range(NROWBUF):
        u = 2 + i * NROWBUF + k
        k3 = (2 + k) % NROWBUF
        ke_even = (2 + k) % NEBUF
        ke_odd = (5 + k) % NEBUF
        is_even = lax.rem(i, 2) == 0

        def do_step(k6):
          lim_e = NCHUNK - 1 - 4 - 2 - k
          ke = (k6 + 4) % NEBUF

          @pl.when(i * NROWBUF <= lim_e)
          def _():
            pltpu.make_async_copy(z1_hbm.at[pl.ds(0, C)], vbuf[ke],
                                  vsem[ke]).wait()
            if use_num:
              pltpu.make_async_copy(z1_hbm.at[pl.ds(0, C)], pbuf[ke],
                                    psem[ke]).wait()
            pltpu.async_copy(edata_hbm.at[wid, u + 4], ebuf[ke], esem[ke])

          lim_g = NCHUNK - 1 - 2 - 2 - k
          kg = (k3 + 2) % NROWBUF
          keg = (k6 + 2) % NEBUF

          @pl.when(i * NROWBUF <= lim_g)
          def _():
            pltpu.make_async_copy(edata_hbm.at[0, 0], ebuf[keg],
                                  esem[keg]).wait()
            pltpu.make_async_copy(z2_hbm.at[pl.ds(0, C)], rows[kg],
                                  ssem[kg]).wait()
            pltpu.async_copy(x_hbm.at[ebuf[keg].at[0]], rows[kg], gsem[kg])

          pltpu.make_async_copy(z2_hbm.at[pl.ds(0, C)], rows[k3],
                                gsem[k3]).wait()
          scaled_process(u, k3, k6)
          issue_scatters(u, k3, k6)

        @pl.when(is_even)
        def _():
          do_step(ke_even)

        @pl.when(jnp.logical_not(is_even))
        def _():
          do_step(ke_odd)
      return carry

    lax.fori_loop(0, (NCHUNK - 2) // NROWBUF, main_body, 0)

    for k in range(NROWBUF):
      pltpu.make_async_copy(z2_hbm.at[pl.ds(0, C)], rows[k], ssem[k]).wait()
    for k in range(NEBUF):
      pltpu.make_async_copy(z1_hbm.at[pl.ds(0, C)], vbuf[k], vsem[k]).wait()
      if use_num:
        pltpu.make_async_copy(z1_hbm.at[pl.ds(0, C)], pbuf[k],
                              psem[k]).wait()
    plsc.subcore_barrier()

    @pl.when(sid < NS - 1)
    def _():
      pltpu.sync_copy(acc_sh.at[rsl], emb_out.at[cid, rsl])

    @pl.when(sid == NS - 1)
    def _():
      pltpu.sync_copy(acc_sh.at[lsl], emb_out.at[cid, lsl])

    pltpu.sync_copy(accn_sh.at[rsl], num_out.at[cid, rsl])
    pltpu.sync_copy(acco_sh.at[rsl], ord_out.at[cid, rsl])

  return spmm


_spmm_l0 = _make_spmm_kernel(1.0, 1.0, False)
_spmm_l1 = _make_spmm_kernel(0.5, 1.0, True)
_spmm_l2 = _make_spmm_kernel(0.5, 0.25, True)


def _combine_body(level, ep_ref, ns_ref, os_ref, pe_ref, pn_ref, po_ref,
                  emb_o, num_o, ord_o):
  sp_e = ep_ref[0] + ep_ref[1]
  sp_o = (os_ref[0] + os_ref[1]).reshape(BLK, 1)
  if level == 0:
    emb_o[...] = sp_e - pe_ref[...]
    num_o[...] = sp_o
  else:
    sp_n = (ns_ref[0] + ns_ref[1]).reshape(BLK, 1)
    po = po_ref[...]
    emb_o[...] = sp_e - (1.0 + po) * pe_ref[...]
    num_o[...] = sp_n - pn_ref[...] - po
  ord_o[...] = sp_o


def _combine(level, emb_part, num_st, ord_st, prev_emb, prev_num, prev_ord):
  grid = (NPAD // BLK,)
  return pl.pallas_call(
      functools.partial(_combine_body, level),
      grid=grid,
      in_specs=[
          pl.BlockSpec((NC, BLK, D), lambda i: (0, i, 0)),
          pl.BlockSpec((NC, BLK), lambda i: (0, i)),
          pl.BlockSpec((NC, BLK), lambda i: (0, i)),
          pl.BlockSpec((BLK, D), lambda i: (i, 0)),
          pl.BlockSpec((BLK, 1), lambda i: (i, 0)),
          pl.BlockSpec((BLK, 1), lambda i: (i, 0)),
      ],
      out_specs=[
          pl.BlockSpec((BLK, D), lambda i: (i, 0)),
          pl.BlockSpec((BLK, 1), lambda i: (i, 0)),
          pl.BlockSpec((BLK, 1), lambda i: (i, 0)),
      ],
      out_shape=[
          jax.ShapeDtypeStruct((NPAD, D), jnp.float32),
          jax.ShapeDtypeStruct((NPAD, 1), jnp.float32),
          jax.ShapeDtypeStruct((NPAD, 1), jnp.float32),
      ],
  )(emb_part, num_st, ord_st, prev_emb, prev_num, prev_ord)


def _scores_body(e1, e2, e3, n1, n2, n3, emb, un, out):
  i = pl.program_id(0)
  sum_e = e1[...] + e2[...] + e3[...]
  sum_n = n1[...] + n2[...] + n3[...]
  sub = sum_e / (sum_n + 1e-08)
  nrm = jnp.sqrt(jnp.sum(sub * sub, axis=1, keepdims=True))
  sub = sub / jnp.maximum(nrm, 1e-12)
  eb = emb[...]
  enrm = jnp.sqrt(jnp.sum(eb * eb, axis=1, keepdims=True))
  eb = eb / jnp.maximum(enrm, 1e-12)
  dot = jnp.sum(sub * eb, axis=1, keepdims=True)
  noise = -jnp.log(-jnp.log(un[...]))
  rid = i * BLK + lax.broadcasted_iota(jnp.int32, (BLK, 1), 0)
  out[...] = jnp.where(rid < N, dot + noise, jnp.float32(-1e30))


def _scores(e1, e2, e3, n1, n2, n3, emb, un):
  grid = (NPAD // BLK,)
  bs_e = pl.BlockSpec((BLK, D), lambda i: (i, 0))
  bs_n = pl.BlockSpec((BLK, 1), lambda i: (i, 0))
  return pl.pallas_call(
      _scores_body,
      grid=grid,
      in_specs=[bs_e, bs_e, bs_e, bs_n, bs_n, bs_n, bs_e, bs_n],
      out_specs=bs_n,
      out_shape=jax.ShapeDtypeStruct((NPAD, 1), jnp.float32),
  )(e1, e2, e3, n1, n2, n3, emb, un)


def _topk_body(x_ref, out_ref):
  rows, cols = x_ref.shape
  ri = lax.broadcasted_iota(jnp.int32, (rows, cols), 0)
  ci = lax.broadcasted_iota(jnp.int32, (rows, cols), 1)
  gidx = ri * cols + ci

  def it(i, x):
    m = jnp.max(x)
    cand = jnp.where(x == m, gidx, jnp.int32(2**30))
    j = jnp.min(cand)
    out_ref[pl.ds(i, 1), :] = jnp.full((1, 1), j, jnp.int32)
    return jnp.where(gidx == j, jnp.float32(-3e38), x)

  lax.fori_loop(0, NUM_CAND, it, x_ref[...])


def _topk(scores_2d):
  return pl.pallas_call(
      _topk_body,
      out_shape=jax.ShapeDtypeStruct((NUM_CAND, 1), jnp.int32),
  )(scores_2d)


def kernel(embeds, edge_index, edge_vals):
  dst = edge_index[0]
  src = edge_index[1]

  key = jax.random.key(42)
  key, sub = jax.random.split(key)
  u1 = jax.random.uniform(sub, (E,), dtype=jnp.float32)
  key, sub = jax.random.split(key)
  u2 = jax.random.uniform(sub, (E,), dtype=jnp.float32)
  nkey = jax.random.key(7)
  un = jax.random.uniform(nkey, (N,), dtype=jnp.float32,
                          minval=1e-20, maxval=1.0)
  un_pad = jnp.pad(un, (0, NPAD - N), constant_values=0.5).reshape(NPAD, 1)

  embeds_pad = jnp.pad(embeds, ((0, NPAD - N), (0, 0)))
  z2 = jnp.zeros((NPAD, D), jnp.float32)
  z1 = jnp.zeros((NPAD,), jnp.float32)
  zcol = jnp.zeros((NPAD, 1), jnp.float32)

  edata = jnp.stack([
      src, dst,
      lax.bitcast_convert_type(edge_vals, jnp.int32),
      lax.bitcast_convert_type(u1, jnp.int32),
      lax.bitcast_convert_type(u2, jnp.int32),
  ])
  edata = edata.reshape(5, NW, NCHUNK, C).transpose(1, 2, 0, 3)

  ep0, _, os0 = _spmm_l0(embeds_pad, z1, edata, z2, z1)
  emb1, num1, ord0 = _combine(0, ep0, os0, os0, embeds_pad, zcol, zcol)

  ep1, ns1, os1 = _spmm_l1(emb1, num1[:, 0], edata, z2, z1)
  emb2, num2, ord1 = _combine(1, ep1, ns1, os1, emb1, num1, ord0)

  ep2, ns2, os2 = _spmm_l2(emb2, num2[:, 0], edata, z2, z1)
  emb3, num3, _ = _combine(1, ep2, ns2, os2, emb2, num2, ord1)

  scores_pad = _scores(emb1, emb2, emb3, num1, num2, num3, embeds_pad,
                       un_pad)
  cand = _topk(scores_pad.reshape(8, NPAD // 8))

  return scores_pad[:N, 0], cand[:, 0]


# speedup vs baseline: 11.4504x; 1.2000x over previous
import functools

import jax
import jax.numpy as jnp
from jax import lax
from jax.experimental import pallas as pl
from jax.experimental.pallas import tpu as pltpu
from jax.experimental.pallas import tpu_sc as plsc

N = 10000
E = 320000
D = 128
NPAD = 10240
NC = 2
NS = 16
NW = NC * NS
EPT = E // NW
C = 80
NCHUNK = EPT // C
RPT = NPAD // NS
NUM_CAND = 500
BLK = 1024

NROWBUF = 3
NEBUF = 6
G16 = C // 16


def _make_spmm_kernel(p1: float, p2: float, use_num: bool):
  mesh = plsc.VectorSubcoreMesh(core_axis_name="c", subcore_axis_name="s")
  out_type = (
      jax.ShapeDtypeStruct((NC, NPAD, D), jnp.float32),
      jax.ShapeDtypeStruct((NC, NPAD), jnp.float32),
      jax.ShapeDtypeStruct((NC, NPAD), jnp.float32),
  )
  scratch = (
      [pltpu.VMEM_SHARED((N, D), jnp.float32)]
      + [pltpu.VMEM_SHARED((NPAD,), jnp.float32)] * 2
      + [pltpu.VMEM((NPAD,), jnp.float32)]
      + [pltpu.VMEM((5, C), jnp.int32)] * NEBUF
      + [pltpu.VMEM((C,), jnp.float32)] * NEBUF
      + [pltpu.VMEM((C,), jnp.float32)] * NEBUF
      + [pltpu.VMEM((C, D), jnp.float32)] * NROWBUF
      + [pltpu.SemaphoreType.DMA] * (3 * NEBUF + 2 * NROWBUF)
  )

  @functools.partial(
      pl.kernel, mesh=mesh, out_type=out_type, scratch_types=scratch,
      compiler_params=pltpu.CompilerParams(needs_layout_passes=False))
  def spmm(x_hbm, numvec_hbm, edata_hbm, z2_hbm, z1_hbm,
           emb_out, num_out, ord_out,
           acc_sh, accn_sh, acco_sh, numvec_v, *rest):
    ebuf = rest[:NEBUF]
    vbuf = rest[NEBUF:2 * NEBUF]
    pbuf = rest[2 * NEBUF:3 * NEBUF]
    rows = rest[3 * NEBUF:3 * NEBUF + NROWBUF]
    sems = rest[3 * NEBUF + NROWBUF:]
    esem = sems[:NEBUF]
    vsem = sems[NEBUF:2 * NEBUF]
    psem = sems[2 * NEBUF:3 * NEBUF]
    gsem = sems[3 * NEBUF:3 * NEBUF + NROWBUF]
    ssem = sems[3 * NEBUF + NROWBUF:]
    cid = lax.axis_index("c")
    sid = lax.axis_index("s")
    wid = sid * NC + cid

    rsl = pl.ds(sid * RPT, RPT)
    lsl = pl.ds((NS - 1) * RPT, N - (NS - 1) * RPT)

    @pl.when(sid < NS - 1)
    def _():
      pltpu.sync_copy(z2_hbm.at[rsl], acc_sh.at[rsl])

    @pl.when(sid == NS - 1)
    def _():
      pltpu.sync_copy(z2_hbm.at[lsl], acc_sh.at[lsl])

    pltpu.sync_copy(z1_hbm.at[rsl], accn_sh.at[rsl])
    pltpu.sync_copy(z1_hbm.at[rsl], acco_sh.at[rsl])
    if use_num:
      pltpu.sync_copy(numvec_hbm, numvec_v)
    plsc.subcore_barrier()

    def scaled_process(t, k3, k6):
      def gbody(g, carry):
        gsl = pl.ds(g * 16, 16)
        v16 = plsc.bitcast(ebuf[k6][2, gsl], jnp.float32)
        one = jnp.ones((16,), jnp.float32)
        zero = jnp.zeros((16,), jnp.float32)
        if p1 < 1.0:
          u1v = plsc.bitcast(ebuf[k6][3, gsl], jnp.float32)
          v16 = v16 * jnp.where(u1v >= (1.0 - p1), one, zero)
        if p2 < 1.0:
          u2v = plsc.bitcast(ebuf[k6][4, gsl], jnp.float32)
          v16 = v16 * jnp.where(u2v >= (1.0 - p2), one, zero)
        vbuf[k6][gsl] = v16
        if use_num:
          s16 = ebuf[k6][0, gsl]
          g16 = plsc.load_gather(numvec_v, [s16])
          pbuf[k6][gsl] = g16 * v16
        for e in range(16):
          row = g * 16 + e
          vs = jnp.full((16,), v16[e], jnp.float32)
          for j in range(D // 16):
            fsl = pl.ds(j * 16, 16)
            rows[k3][row, fsl] = rows[k3][row, fsl] * vs
        return carry

      lax.fori_loop(0, G16, gbody, 0)

    def issue_scatters(t, k3, k6):
      idx = ebuf[k6].at[1]
      pltpu.async_copy(rows[k3], acc_sh.at[idx], ssem[k3], add=True)
      pltpu.async_copy(vbuf[k6], acco_sh.at[idx], vsem[k6], add=True)
      if use_num:
        pltpu.async_copy(pbuf[k6], accn_sh.at[idx], psem[k6], add=True)

    def step(u, k3, k6, *, e_wait_vp, e_issue, g_swait, g_issue):
      if e_issue:
        ke = (k6 + 4) % NEBUF
        if e_wait_vp:
          pltpu.make_async_copy(z1_hbm.at[pl.ds(0, C)], vbuf[ke],
                                vsem[ke]).wait()
          if use_num:
            pltpu.make_async_copy(z1_hbm.at[pl.ds(0, C)], pbuf[ke],
                                  psem[ke]).wait()
        pltpu.async_copy(edata_hbm.at[wid, u + 4], ebuf[ke], esem[ke])
      if g_issue:
        kg = (k3 + 2) % NROWBUF
        keg = (k6 + 2) % NEBUF
        pltpu.make_async_copy(edata_hbm.at[0, 0], ebuf[keg],
                              esem[keg]).wait()
        if g_swait:
          pltpu.make_async_copy(z2_hbm.at[pl.ds(0, C)], rows[kg],
                                ssem[kg]).wait()
        pltpu.async_copy(x_hbm.at[ebuf[keg].at[0]], rows[kg], gsem[kg])
      pltpu.make_async_copy(z2_hbm.at[pl.ds(0, C)], rows[k3],
                            gsem[k3]).wait()
      scaled_process(u, k3, k6)
      issue_scatters(u, k3, k6)

    for u0 in range(4):
      pltpu.async_copy(edata_hbm.at[wid, u0], ebuf[u0], esem[u0])
    for u0 in range(2):
      pltpu.make_async_copy(edata_hbm.at[0, 0], ebuf[u0], esem[u0]).wait()
      pltpu.async_copy(x_hbm.at[ebuf[u0].at[0]], rows[u0], gsem[u0])

    step(0, 0, 0, e_wait_vp=False, e_issue=True, g_swait=False, g_issue=True)
    step(1, 1, 1, e_wait_vp=False, e_issue=True, g_swait=True, g_issue=True)

    def main_body(i, carry):
      for k in range(NROWBUF):
        u = 2 + i * NROWBUF + k
        k3 = (2 + k) % NROWBUF
        ke_even = (2 + k) % NEBUF
        ke_odd = (5 + k) % NEBUF
        is_even = lax.rem(i, 2) == 0

        def do_step(k6):
          lim_e = NCHUNK - 1 - 4 - 2 - k
          ke = (k6 + 4) % NEBUF

          @pl.when(i * NROWBUF <= lim_e)
          def _():
            pltpu.make_async_copy(z1_hbm.at[pl.ds(0, C)], vbuf[ke],
                                  vsem[ke]).wait()
            if use_num:
              pltpu.make_async_copy(z1_hbm.at[pl.ds(0, C)], pbuf[ke],
                                    psem[ke]).wait()
            pltpu.async_copy(edata_hbm.at[wid, u + 4], ebuf[ke], esem[ke])

          lim_g = NCHUNK - 1 - 2 - 2 - k
          kg = (k3 + 2) % NROWBUF
          keg = (k6 + 2) % NEBUF

          @pl.when(i * NROWBUF <= lim_g)
          def _():
            pltpu.make_async_copy(edata_hbm.at[0, 0], ebuf[keg],
                                  esem[keg]).wait()
            pltpu.make_async_copy(z2_hbm.at[pl.ds(0, C)], rows[kg],
                                  ssem[kg]).wait()
            pltpu.async_copy(x_hbm.at[ebuf[keg].at[0]], rows[kg], gsem[kg])

          pltpu.make_async_copy(z2_hbm.at[pl.ds(0, C)], rows[k3],
                                gsem[k3]).wait()
          scaled_process(u, k3, k6)
          issue_scatters(u, k3, k6)

        @pl.when(is_even)
        def _():
          do_step(ke_even)

        @pl.when(jnp.logical_not(is_even))
        def _():
          do_step(ke_odd)
      return carry

    lax.fori_loop(0, (NCHUNK - 2) // NROWBUF, main_body, 0)

    for k in range(NROWBUF):
      pltpu.make_async_copy(z2_hbm.at[pl.ds(0, C)], rows[k], ssem[k]).wait()
    for k in range(NEBUF):
      pltpu.make_async_copy(z1_hbm.at[pl.ds(0, C)], vbuf[k], vsem[k]).wait()
      if use_num:
        pltpu.make_async_copy(z1_hbm.at[pl.ds(0, C)], pbuf[k],
                              psem[k]).wait()
    plsc.subcore_barrier()

    @pl.when(sid < NS - 1)
    def _():
      pltpu.sync_copy(acc_sh.at[rsl], emb_out.at[cid, rsl])

    @pl.when(sid == NS - 1)
    def _():
      pltpu.sync_copy(acc_sh.at[lsl], emb_out.at[cid, lsl])

    pltpu.sync_copy(accn_sh.at[rsl], num_out.at[cid, rsl])
    pltpu.sync_copy(acco_sh.at[rsl], ord_out.at[cid, rsl])

  return spmm


_spmm_l0 = _make_spmm_kernel(1.0, 1.0, False)
_spmm_l1 = _make_spmm_kernel(0.5, 1.0, True)
_spmm_l2 = _make_spmm_kernel(0.5, 0.25, True)


def _combine_body(level, ep_ref, ns_ref, os_ref, pe_ref, pn_ref, po_ref,
                  emb_o, num_o, ord_o):
  sp_e = ep_ref[0] + ep_ref[1]
  sp_o = (os_ref[0] + os_ref[1]).reshape(BLK, 1)
  if level == 0:
    emb_o[...] = sp_e - pe_ref[...]
    num_o[...] = sp_o
  else:
    sp_n = (ns_ref[0] + ns_ref[1]).reshape(BLK, 1)
    po = po_ref[...]
    emb_o[...] = sp_e - (1.0 + po) * pe_ref[...]
    num_o[...] = sp_n - pn_ref[...] - po
  ord_o[...] = sp_o


def _combine(level, emb_part, num_st, ord_st, prev_emb, prev_num, prev_ord):
  grid = (NPAD // BLK,)
  return pl.pallas_call(
      functools.partial(_combine_body, level),
      grid=grid,
      in_specs=[
          pl.BlockSpec((NC, BLK, D), lambda i: (0, i, 0)),
          pl.BlockSpec((NC, BLK), lambda i: (0, i)),
          pl.BlockSpec((NC, BLK), lambda i: (0, i)),
          pl.BlockSpec((BLK, D), lambda i: (i, 0)),
          pl.BlockSpec((BLK, 1), lambda i: (i, 0)),
          pl.BlockSpec((BLK, 1), lambda i: (i, 0)),
      ],
      out_specs=[
          pl.BlockSpec((BLK, D), lambda i: (i, 0)),
          pl.BlockSpec((BLK, 1), lambda i: (i, 0)),
          pl.BlockSpec((BLK, 1), lambda i: (i, 0)),
      ],
      out_shape=[
          jax.ShapeDtypeStruct((NPAD, D), jnp.float32),
          jax.ShapeDtypeStruct((NPAD, 1), jnp.float32),
          jax.ShapeDtypeStruct((NPAD, 1), jnp.float32),
      ],
  )(emb_part, num_st, ord_st, prev_emb, prev_num, prev_ord)


def _scores_body(e1, e2, e3, n1, n2, n3, emb, un, out):
  i = pl.program_id(0)
  sum_e = e1[...] + e2[...] + e3[...]
  sum_n = n1[...] + n2[...] + n3[...]
  sub = sum_e / (sum_n + 1e-08)
  nrm = jnp.sqrt(jnp.sum(sub * sub, axis=1, keepdims=True))
  sub = sub / jnp.maximum(nrm, 1e-12)
  eb = emb[...]
  enrm = jnp.sqrt(jnp.sum(eb * eb, axis=1, keepdims=True))
  eb = eb / jnp.maximum(enrm, 1e-12)
  dot = jnp.sum(sub * eb, axis=1, keepdims=True)
  noise = -jnp.log(-jnp.log(un[...]))
  rid = i * BLK + lax.broadcasted_iota(jnp.int32, (BLK, 1), 0)
  out[...] = jnp.where(rid < N, dot + noise, jnp.float32(-1e30))


def _scores(e1, e2, e3, n1, n2, n3, emb, un):
  grid = (NPAD // BLK,)
  bs_e = pl.BlockSpec((BLK, D), lambda i: (i, 0))
  bs_n = pl.BlockSpec((BLK, 1), lambda i: (i, 0))
  return pl.pallas_call(
      _scores_body,
      grid=grid,
      in_specs=[bs_e, bs_e, bs_e, bs_n, bs_n, bs_n, bs_e, bs_n],
      out_specs=bs_n,
      out_shape=jax.ShapeDtypeStruct((NPAD, 1), jnp.float32),
  )(e1, e2, e3, n1, n2, n3, emb, un)


def _topk_body(x_ref, out_ref):
  rows, cols = x_ref.shape
  x = x_ref[...]
  ri = lax.broadcasted_iota(jnp.int32, (rows, cols), 0)
  ci = lax.broadcasted_iota(jnp.int32, (rows, cols), 1)
  gidx = ri * cols + ci

  minint = jnp.int32(-2**31)
  b = lax.bitcast_convert_type(x, jnp.int32)
  ks = jnp.where(b < 0, jnp.bitwise_xor(jnp.bitwise_not(b), minint), b)

  def tbody(bi, prefix_u):
    thr_u = jnp.bitwise_or(prefix_u, lax.shift_left(jnp.int32(1), 31 - bi))
    thr_s = jnp.bitwise_xor(thr_u, minint)
    cnt = jnp.sum(jnp.where(ks >= thr_s, jnp.float32(1.0), jnp.float32(0.0)))
    return jnp.where(cnt >= jnp.float32(NUM_CAND), thr_u, prefix_u)

  k_s = jnp.bitwise_xor(lax.fori_loop(0, 32, tbody, jnp.int32(0)), minint)

  one = jnp.float32(1.0)
  zero = jnp.float32(0.0)
  ri2 = lax.broadcasted_iota(jnp.int32, (cols, cols), 0)
  ci2 = lax.broadcasted_iota(jnp.int32, (cols, cols), 1)
  t_incl = jnp.where(ri2 <= ci2, one, zero)
  ri8 = lax.broadcasted_iota(jnp.int32, (rows, rows), 0)
  ci8 = lax.broadcasted_iota(jnp.int32, (rows, rows), 1)
  t8_strict = jnp.where(ci8 < ri8, one, zero)

  def prefix_rank(flag_f):
    s1 = jnp.dot(flag_f, t_incl, preferred_element_type=jnp.float32, precision=lax.Precision.HIGHEST)
    rowtot = s1[:, cols - 1:cols]
    rowoff = jnp.dot(t8_strict, rowtot, preferred_element_type=jnp.float32, precision=lax.Precision.HIGHEST)
    return rowoff + s1 - flag_f

  gt = ks > k_s
  eq = ks == k_s
  gt_f = jnp.where(gt, one, zero)
  eq_f = jnp.where(eq, one, zero)
  cnt_gt = jnp.sum(gt_f)
  rank_eq = prefix_rank(eq_f)
  take = jnp.logical_and(eq, rank_eq < (jnp.float32(NUM_CAND) - cnt_gt))
  msel = jnp.logical_or(gt, take)

  m_f = jnp.where(msel, one, zero)
  rsel = prefix_rank(m_f)

  nb = (rows * cols) // 1024
  rself = rsel.reshape(nb, 1024)
  mself = m_f.reshape(nb, 1024)
  xf = x.reshape(nb, 1024)
  gf = gidx.astype(jnp.float32).reshape(nb, 1024)
  pio = lax.broadcasted_iota(jnp.int32, (512, 1024), 0).astype(jnp.float32)
  acc = jnp.zeros((512, 2), jnp.float32)
  for blk in range(nb):
    d = rself[blk][None, :] - pio
    pb = jnp.where(d == 0.0, jnp.float32(1.0), jnp.float32(0.0))
    pb = pb * mself[blk][None, :]
    xb = jnp.stack([xf[blk], gf[blk]], axis=1)
    acc = acc + jnp.dot(pb, xb, preferred_element_type=jnp.float32, precision=lax.Precision.HIGHEST)
  cscore = acc[:, 0:1]
  cidx = acc[:, 1:2]

  rio = lax.broadcasted_iota(jnp.int32, (512, 512), 0).astype(jnp.float32)
  qio = lax.broadcasted_iota(jnp.int32, (512, 512), 1).astype(jnp.float32)
  valid_r = jnp.where(rio < float(NUM_CAND), jnp.float32(1.0),
                      jnp.float32(0.0))
  cs_r = cscore
  cs_q = cscore.reshape(1, 512)
  ci_r = cidx
  ci_q = cidx.reshape(1, 512)
  before = jnp.logical_or(
      cs_r > cs_q, jnp.logical_and(cs_r == cs_q, ci_r < ci_q))
  bf = jnp.where(before, jnp.float32(1.0), jnp.float32(0.0)) * valid_r
  pos = jnp.sum(bf, axis=0)

  pos_q = pos.reshape(1, 512)
  onehot = jnp.where(
      jnp.logical_and(rio == pos_q, qio < float(NUM_CAND)),
      jnp.float32(1.0), jnp.float32(0.0))
  out = jnp.dot(onehot, cidx, preferred_element_type=jnp.float32, precision=lax.Precision.HIGHEST)
  out_ref[...] = out[:NUM_CAND].astype(jnp.int32)


def _topk(scores_2d):
  return pl.pallas_call(
      _topk_body,
      out_shape=jax.ShapeDtypeStruct((NUM_CAND, 1), jnp.int32),
  )(scores_2d)


def kernel(embeds, edge_index, edge_vals):
  dst = edge_index[0]
  src = edge_index[1]

  key = jax.random.key(42)
  key, sub = jax.random.split(key)
  u1 = jax.random.uniform(sub, (E,), dtype=jnp.float32)
  key, sub = jax.random.split(key)
  u2 = jax.random.uniform(sub, (E,), dtype=jnp.float32)
  nkey = jax.random.key(7)
  un = jax.random.uniform(nkey, (N,), dtype=jnp.float32,
                          minval=1e-20, maxval=1.0)
  un_pad = jnp.pad(un, (0, NPAD - N), constant_values=0.5).reshape(NPAD, 1)

  embeds_pad = jnp.pad(embeds, ((0, NPAD - N), (0, 0)))
  z2 = jnp.zeros((NPAD, D), jnp.float32)
  z1 = jnp.zeros((NPAD,), jnp.float32)
  zcol = jnp.zeros((NPAD, 1), jnp.float32)

  edata = jnp.stack([
      src, dst,
      lax.bitcast_convert_type(edge_vals, jnp.int32),
      lax.bitcast_convert_type(u1, jnp.int32),
      lax.bitcast_convert_type(u2, jnp.int32),
  ])
  edata = edata.reshape(5, NW, NCHUNK, C).transpose(1, 2, 0, 3)

  ep0, _, os0 = _spmm_l0(embeds_pad, z1, edata, z2, z1)
  emb1, num1, ord0 = _combine(0, ep0, os0, os0, embeds_pad, zcol, zcol)

  ep1, ns1, os1 = _spmm_l1(emb1, num1[:, 0], edata, z2, z1)
  emb2, num2, ord1 = _combine(1, ep1, ns1, os1, emb1, num1, ord0)

  ep2, ns2, os2 = _spmm_l2(emb2, num2[:, 0], edata, z2, z1)
  emb3, num3, _ = _combine(1, ep2, ns2, os2, emb2, num2, ord1)

  scores_pad = _scores(emb1, emb2, emb3, num1, num2, num3, embeds_pad,
                       un_pad)
  cand = _topk(scores_pad.reshape(8, NPAD // 8))

  return scores_pad[:N, 0], cand[:, 0]


# speedup vs baseline: 11.7274x; 1.0242x over previous
import functools

import jax
import jax.numpy as jnp
from jax import lax
from jax.experimental import pallas as pl
from jax.experimental.pallas import tpu as pltpu
from jax.experimental.pallas import tpu_sc as plsc

N = 10000
E = 320000
D = 128
NPAD = 10240
NC = 2
NS = 16
NW = NC * NS
EPT = E // NW
C = 80
NCHUNK = EPT // C
RPT = NPAD // NS
NUM_CAND = 500
BLK = 1024

NROWBUF = 3
NEBUF = 6
G16 = C // 16


def _make_spmm_kernel(p1: float, p2: float, use_num: bool):
  mesh = plsc.VectorSubcoreMesh(core_axis_name="c", subcore_axis_name="s")
  out_type = (
      jax.ShapeDtypeStruct((NC, NPAD, D), jnp.float32),
      jax.ShapeDtypeStruct((NC, NPAD), jnp.float32),
      jax.ShapeDtypeStruct((NC, NPAD), jnp.float32),
  )
  scratch = (
      [pltpu.VMEM_SHARED((N, D), jnp.float32)]
      + [pltpu.VMEM_SHARED((NPAD,), jnp.float32)] * 2
      + [pltpu.VMEM((NPAD,), jnp.float32)]
      + [pltpu.VMEM((5, C), jnp.int32)] * NEBUF
      + [pltpu.VMEM((C,), jnp.float32)] * NEBUF
      + [pltpu.VMEM((C,), jnp.float32)] * NEBUF
      + [pltpu.VMEM((C, D), jnp.float32)] * NROWBUF
      + [pltpu.SemaphoreType.DMA] * (3 * NEBUF + 2 * NROWBUF)
  )

  @functools.partial(
      pl.kernel, mesh=mesh, out_type=out_type, scratch_types=scratch,
      compiler_params=pltpu.CompilerParams(needs_layout_passes=False))
  def spmm(x_hbm, numvec_hbm, edata_hbm, z2_hbm, z1_hbm,
           emb_out, num_out, ord_out,
           acc_sh, accn_sh, acco_sh, numvec_v, *rest):
    ebuf = rest[:NEBUF]
    vbuf = rest[NEBUF:2 * NEBUF]
    pbuf = rest[2 * NEBUF:3 * NEBUF]
    rows = rest[3 * NEBUF:3 * NEBUF + NROWBUF]
    sems = rest[3 * NEBUF + NROWBUF:]
    esem = sems[:NEBUF]
    vsem = sems[NEBUF:2 * NEBUF]
    psem = sems[2 * NEBUF:3 * NEBUF]
    gsem = sems[3 * NEBUF:3 * NEBUF + NROWBUF]
    ssem = sems[3 * NEBUF + NROWBUF:]
    cid = lax.axis_index("c")
    sid = lax.axis_index("s")
    wid = sid * NC + cid

    rsl = pl.ds(sid * RPT, RPT)
    lsl = pl.ds((NS - 1) * RPT, N - (NS - 1) * RPT)

    @pl.when(sid < NS - 1)
    def _():
      pltpu.sync_copy(z2_hbm.at[rsl], acc_sh.at[rsl])

    @pl.when(sid == NS - 1)
    def _():
      pltpu.sync_copy(z2_hbm.at[lsl], acc_sh.at[lsl])

    pltpu.sync_copy(z1_hbm.at[rsl], accn_sh.at[rsl])
    pltpu.sync_copy(z1_hbm.at[rsl], acco_sh.at[rsl])
    if use_num:
      pltpu.sync_copy(numvec_hbm, numvec_v)
    plsc.subcore_barrier()

    def scaled_process(t, k3, k6):
      def gbody(g, carry):
        gsl = pl.ds(g * 16, 16)
        v16 = plsc.bitcast(ebuf[k6][2, gsl], jnp.float32)
        one = jnp.ones((16,), jnp.float32)
        zero = jnp.zeros((16,), jnp.float32)
        if p1 < 1.0:
          u1v = plsc.bitcast(ebuf[k6][3, gsl], jnp.float32)
          v16 = v16 * jnp.where(u1v >= (1.0 - p1), one, zero)
        if p2 < 1.0:
          u2v = plsc.bitcast(ebuf[k6][4, gsl], jnp.float32)
          v16 = v16 * jnp.where(u2v >= (1.0 - p2), one, zero)
        vbuf[k6][gsl] = v16
        if use_num:
          s16 = ebuf[k6][0, gsl]
          g16 = plsc.load_gather(numvec_v, [s16])
          pbuf[k6][gsl] = g16 * v16
        for e in range(16):
          row = g * 16 + e
          vs = jnp.full((16,), v16[e], jnp.float32)
          for j in range(D // 16):
            fsl = pl.ds(j * 16, 16)
            rows[k3][row, fsl] = rows[k3][row, fsl] * vs
        return carry

      lax.fori_loop(0, G16, gbody, 0)

    def issue_scatters(t, k3, k6):
      idx = ebuf[k6].at[1]
      pltpu.async_copy(rows[k3], acc_sh.at[idx], ssem[k3], add=True)
      pltpu.async_copy(vbuf[k6], acco_sh.at[idx], vsem[k6], add=True)
      if use_num:
        pltpu.async_copy(pbuf[k6], accn_sh.at[idx], psem[k6], add=True)

    def step(u, k3, k6, *, e_wait_vp, e_issue, g_swait, g_issue):
      if e_issue:
        ke = (k6 + 4) % NEBUF
        if e_wait_vp:
          pltpu.make_async_copy(z1_hbm.at[pl.ds(0, C)], vbuf[ke],
                                vsem[ke]).wait()
          if use_num:
            pltpu.make_async_copy(z1_hbm.at[pl.ds(0, C)], pbuf[ke],
                                  psem[ke]).wait()
        pltpu.async_copy(edata_hbm.at[wid, u + 4], ebuf[ke], esem[ke])
      if g_issue:
        kg = (k3 + 2) % NROWBUF
        keg = (k6 + 2) % NEBUF
        pltpu.make_async_copy(edata_hbm.at[0, 0], ebuf[keg],
                              esem[keg]).wait()
        if g_swait:
          pltpu.make_async_copy(z2_hbm.at[pl.ds(0, C)], rows[kg],
                                ssem[kg]).wait()
        pltpu.async_copy(x_hbm.at[ebuf[keg].at[0]], rows[kg], gsem[kg])
      pltpu.make_async_copy(z2_hbm.at[pl.ds(0, C)], rows[k3],
                            gsem[k3]).wait()
      scaled_process(u, k3, k6)
      issue_scatters(u, k3, k6)

    for u0 in range(4):
      pltpu.async_copy(edata_hbm.at[wid, u0], ebuf[u0], esem[u0])
    for u0 in range(2):
      pltpu.make_async_copy(edata_hbm.at[0, 0], ebuf[u0], esem[u0]).wait()
      pltpu.async_copy(x_hbm.at[ebuf[u0].at[0]], rows[u0], gsem[u0])

    step(0, 0, 0, e_wait_vp=False, e_issue=True, g_swait=False, g_issue=True)
    step(1, 1, 1, e_wait_vp=False, e_issue=True, g_swait=True, g_issue=True)

    def main_body(i, carry):
      for k in range(NROWBUF):
        u = 2 + i * NROWBUF + k
        k3 = (2 + k) % NROWBUF
        ke_even = (2 + k) % NEBUF
        ke_odd = (5 + k) % NEBUF
        is_even = lax.rem(i, 2) == 0

        def do_step(k6):
          lim_e = NCHUNK - 1 - 4 - 2 - k
          ke = (k6 + 4) % NEBUF

          @pl.when(i * NROWBUF <= lim_e)
          def _():
            pltpu.make_async_copy(z1_hbm.at[pl.ds(0, C)], vbuf[ke],
                                  vsem[ke]).wait()
            if use_num:
              pltpu.make_async_copy(z1_hbm.at[pl.ds(0, C)], pbuf[ke],
                                    psem[ke]).wait()
            pltpu.async_copy(edata_hbm.at[wid, u + 4], ebuf[ke], esem[ke])

          lim_g = NCHUNK - 1 - 2 - 2 - k
          kg = (k3 + 2) % NROWBUF
          keg = (k6 + 2) % NEBUF

          @pl.when(i * NROWBUF <= lim_g)
          def _():
            pltpu.make_async_copy(edata_hbm.at[0, 0], ebuf[keg],
                                  esem[keg]).wait()
            pltpu.make_async_copy(z2_hbm.at[pl.ds(0, C)], rows[kg],
                                  ssem[kg]).wait()
            pltpu.async_copy(x_hbm.at[ebuf[keg].at[0]], rows[kg], gsem[kg])

          pltpu.make_async_copy(z2_hbm.at[pl.ds(0, C)], rows[k3],
                                gsem[k3]).wait()
          scaled_process(u, k3, k6)
          issue_scatters(u, k3, k6)

        @pl.when(is_even)
        def _():
          do_step(ke_even)

        @pl.when(jnp.logical_not(is_even))
        def _():
          do_step(ke_odd)
      return carry

    lax.fori_loop(0, (NCHUNK - 2) // NROWBUF, main_body, 0)

    for k in range(NROWBUF):
      pltpu.make_async_copy(z2_hbm.at[pl.ds(0, C)], rows[k], ssem[k]).wait()
    for k in range(NEBUF):
      pltpu.make_async_copy(z1_hbm.at[pl.ds(0, C)], vbuf[k], vsem[k]).wait()
      if use_num:
        pltpu.make_async_copy(z1_hbm.at[pl.ds(0, C)], pbuf[k],
                              psem[k]).wait()
    plsc.subcore_barrier()

    @pl.when(sid < NS - 1)
    def _():
      pltpu.sync_copy(acc_sh.at[rsl], emb_out.at[cid, rsl])

    @pl.when(sid == NS - 1)
    def _():
      pltpu.sync_copy(acc_sh.at[lsl], emb_out.at[cid, lsl])

    pltpu.sync_copy(accn_sh.at[rsl], num_out.at[cid, rsl])
    pltpu.sync_copy(acco_sh.at[rsl], ord_out.at[cid, rsl])

  return spmm


_spmm_l0 = _make_spmm_kernel(1.0, 1.0, False)
_spmm_l1 = _make_spmm_kernel(0.5, 1.0, True)
_spmm_l2 = _make_spmm_kernel(0.5, 0.25, True)


def _combine_body(level, ep_ref, ns_ref, os_ref, pe_ref, pn_ref, po_ref,
                  emb_o, num_o, ord_o):
  sp_e = ep_ref[0] + ep_ref[1]
  sp_o = (os_ref[0] + os_ref[1]).reshape(BLK, 1)
  if level == 0:
    emb_o[...] = sp_e - pe_ref[...]
    num_o[...] = sp_o
  else:
    sp_n = (ns_ref[0] + ns_ref[1]).reshape(BLK, 1)
    po = po_ref[...]
    emb_o[...] = sp_e - (1.0 + po) * pe_ref[...]
    num_o[...] = sp_n - pn_ref[...] - po
  ord_o[...] = sp_o


def _combine(level, emb_part, num_st, ord_st, prev_emb, prev_num, prev_ord):
  grid = (NPAD // BLK,)
  return pl.pallas_call(
      functools.partial(_combine_body, level),
      grid=grid,
      in_specs=[
          pl.BlockSpec((NC, BLK, D), lambda i: (0, i, 0)),
          pl.BlockSpec((NC, BLK), lambda i: (0, i)),
          pl.BlockSpec((NC, BLK), lambda i: (0, i)),
          pl.BlockSpec((BLK, D), lambda i: (i, 0)),
          pl.BlockSpec((BLK, 1), lambda i: (i, 0)),
          pl.BlockSpec((BLK, 1), lambda i: (i, 0)),
      ],
      out_specs=[
          pl.BlockSpec((BLK, D), lambda i: (i, 0)),
          pl.BlockSpec((BLK, 1), lambda i: (i, 0)),
          pl.BlockSpec((BLK, 1), lambda i: (i, 0)),
      ],
      out_shape=[
          jax.ShapeDtypeStruct((NPAD, D), jnp.float32),
          jax.ShapeDtypeStruct((NPAD, 1), jnp.float32),
          jax.ShapeDtypeStruct((NPAD, 1), jnp.float32),
      ],
  )(emb_part, num_st, ord_st, prev_emb, prev_num, prev_ord)


def _scores_body(ep2, ns2, e1, e2, n1, n2, o1, emb, un, out):
  i = pl.program_id(0)
  po = o1[...]
  e3 = (ep2[0] + ep2[1]) - (1.0 + po) * e2[...]
  n3 = (ns2[0] + ns2[1]).reshape(BLK, 1) - n2[...] - po
  sum_e = e1[...] + e2[...] + e3
  sum_n = n1[...] + n2[...] + n3
  sub = sum_e / (sum_n + 1e-08)
  nrm = jnp.sqrt(jnp.sum(sub * sub, axis=1, keepdims=True))
  sub = sub / jnp.maximum(nrm, 1e-12)
  eb = emb[...]
  enrm = jnp.sqrt(jnp.sum(eb * eb, axis=1, keepdims=True))
  eb = eb / jnp.maximum(enrm, 1e-12)
  dot = jnp.sum(sub * eb, axis=1, keepdims=True)
  noise = -jnp.log(-jnp.log(un[...]))
  rid = i * BLK + lax.broadcasted_iota(jnp.int32, (BLK, 1), 0)
  out[...] = jnp.where(rid < N, dot + noise, jnp.float32(-1e30))


def _scores(ep2, ns2, e1, e2, n1, n2, o1, emb, un):
  grid = (NPAD // BLK,)
  bs_e = pl.BlockSpec((BLK, D), lambda i: (i, 0))
  bs_n = pl.BlockSpec((BLK, 1), lambda i: (i, 0))
  return pl.pallas_call(
      _scores_body,
      grid=grid,
      in_specs=[
          pl.BlockSpec((NC, BLK, D), lambda i: (0, i, 0)),
          pl.BlockSpec((NC, BLK), lambda i: (0, i)),
          bs_e, bs_e, bs_n, bs_n, bs_n, bs_e, bs_n,
      ],
      out_specs=bs_n,
      out_shape=jax.ShapeDtypeStruct((NPAD, 1), jnp.float32),
  )(ep2, ns2, e1, e2, n1, n2, o1, emb, un)


def _topk_body(x_ref, out_ref):
  rows, cols = x_ref.shape
  x = x_ref[...]
  ri = lax.broadcasted_iota(jnp.int32, (rows, cols), 0)
  ci = lax.broadcasted_iota(jnp.int32, (rows, cols), 1)
  gidx = ri * cols + ci

  minint = jnp.int32(-2**31)
  b = lax.bitcast_convert_type(x, jnp.int32)
  ks = jnp.where(b < 0, jnp.bitwise_xor(jnp.bitwise_not(b), minint), b)

  def tbody(bi, prefix_u):
    thr_u = jnp.bitwise_or(prefix_u, lax.shift_left(jnp.int32(1), 31 - bi))
    thr_s = jnp.bitwise_xor(thr_u, minint)
    cnt = jnp.sum(jnp.where(ks >= thr_s, jnp.float32(1.0), jnp.float32(0.0)))
    return jnp.where(cnt >= jnp.float32(NUM_CAND), thr_u, prefix_u)

  k_s = jnp.bitwise_xor(lax.fori_loop(0, 32, tbody, jnp.int32(0)), minint)

  one = jnp.float32(1.0)
  zero = jnp.float32(0.0)
  ri2 = lax.broadcasted_iota(jnp.int32, (cols, cols), 0)
  ci2 = lax.broadcasted_iota(jnp.int32, (cols, cols), 1)
  t_incl = jnp.where(ri2 <= ci2, one, zero)
  ri8 = lax.broadcasted_iota(jnp.int32, (rows, rows), 0)
  ci8 = lax.broadcasted_iota(jnp.int32, (rows, rows), 1)
  t8_strict = jnp.where(ci8 < ri8, one, zero)

  def prefix_rank(flag_f):
    s1 = jnp.dot(flag_f, t_incl, preferred_element_type=jnp.float32, precision=lax.Precision.HIGHEST)
    rowtot = s1[:, cols - 1:cols]
    rowoff = jnp.dot(t8_strict, rowtot, preferred_element_type=jnp.float32, precision=lax.Precision.HIGHEST)
    return rowoff + s1 - flag_f

  gt = ks > k_s
  eq = ks == k_s
  gt_f = jnp.where(gt, one, zero)
  eq_f = jnp.where(eq, one, zero)
  cnt_gt = jnp.sum(gt_f)
  rank_eq = prefix_rank(eq_f)
  take = jnp.logical_and(eq, rank_eq < (jnp.float32(NUM_CAND) - cnt_gt))
  msel = jnp.logical_or(gt, take)

  m_f = jnp.where(msel, one, zero)
  rsel = prefix_rank(m_f)

  nb = (rows * cols) // 1024
  rself = rsel.reshape(nb, 1024)
  mself = m_f.reshape(nb, 1024)
  xf = x.reshape(nb, 1024)
  gf = gidx.astype(jnp.float32).reshape(nb, 1024)
  pio = lax.broadcasted_iota(jnp.int32, (512, 1024), 0).astype(jnp.float32)
  acc = jnp.zeros((512, 2), jnp.float32)
  for blk in range(nb):
    d = rself[blk][None, :] - pio
    pb = jnp.where(d == 0.0, jnp.float32(1.0), jnp.float32(0.0))
    pb = pb * mself[blk][None, :]
    xb = jnp.stack([xf[blk], gf[blk]], axis=1)
    acc = acc + jnp.dot(pb, xb, preferred_element_type=jnp.float32, precision=lax.Precision.HIGHEST)
  cscore = acc[:, 0:1]
  cidx = acc[:, 1:2]

  rio = lax.broadcasted_iota(jnp.int32, (512, 512), 0).astype(jnp.float32)
  qio = lax.broadcasted_iota(jnp.int32, (512, 512), 1).astype(jnp.float32)
  valid_r = jnp.where(rio < float(NUM_CAND), jnp.float32(1.0),
                      jnp.float32(0.0))
  cs_r = cscore
  cs_q = cscore.reshape(1, 512)
  ci_r = cidx
  ci_q = cidx.reshape(1, 512)
  before = jnp.logical_or(
      cs_r > cs_q, jnp.logical_and(cs_r == cs_q, ci_r < ci_q))
  bf = jnp.where(before, jnp.float32(1.0), jnp.float32(0.0)) * valid_r
  pos = jnp.sum(bf, axis=0)

  pos_q = pos.reshape(1, 512)
  onehot = jnp.where(
      jnp.logical_and(rio == pos_q, qio < float(NUM_CAND)),
      jnp.float32(1.0), jnp.float32(0.0))
  out = jnp.dot(onehot, cidx, preferred_element_type=jnp.float32, precision=lax.Precision.HIGHEST)
  out_ref[...] = out[:NUM_CAND].astype(jnp.int32)


def _topk(scores_2d):
  return pl.pallas_call(
      _topk_body,
      out_shape=jax.ShapeDtypeStruct((NUM_CAND, 1), jnp.int32),
  )(scores_2d)


def kernel(embeds, edge_index, edge_vals):
  dst = edge_index[0]
  src = edge_index[1]

  key = jax.random.key(42)
  key, sub = jax.random.split(key)
  u1 = jax.random.uniform(sub, (E,), dtype=jnp.float32)
  key, sub = jax.random.split(key)
  u2 = jax.random.uniform(sub, (E,), dtype=jnp.float32)
  nkey = jax.random.key(7)
  un = jax.random.uniform(nkey, (N,), dtype=jnp.float32,
                          minval=1e-20, maxval=1.0)
  un_pad = jnp.pad(un, (0, NPAD - N), constant_values=0.5).reshape(NPAD, 1)

  embeds_pad = jnp.pad(embeds, ((0, NPAD - N), (0, 0)))
  z2 = jnp.zeros((NPAD, D), jnp.float32)
  z1 = jnp.zeros((NPAD,), jnp.float32)
  zcol = jnp.zeros((NPAD, 1), jnp.float32)

  edata = jnp.stack([
      src, dst,
      lax.bitcast_convert_type(edge_vals, jnp.int32),
      lax.bitcast_convert_type(u1, jnp.int32),
      lax.bitcast_convert_type(u2, jnp.int32),
  ])
  edata = edata.reshape(5, NW, NCHUNK, C).transpose(1, 2, 0, 3)

  ep0, _, os0 = _spmm_l0(embeds_pad, z1, edata, z2, z1)
  emb1, num1, ord0 = _combine(0, ep0, os0, os0, embeds_pad, zcol, zcol)

  ep1, ns1, os1 = _spmm_l1(emb1, num1[:, 0], edata, z2, z1)
  emb2, num2, ord1 = _combine(1, ep1, ns1, os1, emb1, num1, ord0)

  ep2, ns2, _ = _spmm_l2(emb2, num2[:, 0], edata, z2, z1)

  scores_pad = _scores(ep2, ns2, emb1, emb2, num1, num2, ord1, embeds_pad,
                       un_pad)
  cand = _topk(scores_pad.reshape(8, NPAD // 8))

  return scores_pad[:N, 0], cand[:, 0]
